# baseline (speedup 1.0000x reference)
import functools

import jax
import jax.numpy as jnp
from jax import lax
from jax.experimental import pallas as pl
from jax.experimental.pallas import tpu as pltpu
from jax.experimental.pallas import tpu_sc as plsc

N = 8
NF = 200
NFP = 208
NJ = 10
L = 16
NCHUNK = NFP // L
PAD_ROWS = 64

_GDN = lax.GatherDimensionNumbers(
    offset_dims=(), collapsed_slice_dims=(0,), start_index_map=(0,))


def _lane_bcast(vec, idx16):
    return lax.gather(vec, idx16[:, None], _GDN, (1,),
                      mode=lax.GatherScatterMode.PROMISE_IN_BOUNDS)


def _splat(vec, j):
    return _lane_bcast(vec, jnp.full((16,), j, jnp.int32))


def _sc_body(x_hbm, wt_hbm, b_hbm, outv_hbm, outi_hbm, outa_hbm,
             x_v, wt_v, b_v, stage_v, all_v, vout_v, iout_v, aout_v, ashared):
    c = lax.axis_index("c")
    s = lax.axis_index("s")
    lane = lax.iota(jnp.int32, 16)
    r = jnp.where(s < N, s, 0)

    pltpu.sync_copy(x_hbm.at[r], x_v)
    pltpu.sync_copy(wt_hbm, wt_v)
    pltpu.sync_copy(b_hbm, b_v)

    xc = [x_v[pl.ds(L * cc, L)] for cc in range(NCHUNK)]
    z = b_v[...]
    for j in range(NJ):
        acc = xc[0] * wt_v[j, pl.ds(0, L)]
        for cc in range(1, NCHUNK):
            acc = acc + xc[cc] * wt_v[j, pl.ds(L * cc, L)]
        z = jnp.where(lane == j, z + jnp.sum(acc), z)

    sig = 1.0 / (1.0 + jnp.exp(-z))
    atrr = jnp.where((lane < NJ) & (s < N), sig, 0.0)
    stage_v[...] = atrr

    pltpu.sync_copy(stage_v, ashared.at[PAD_ROWS + s])
    plsc.subcore_barrier()
    pltpu.sync_copy(ashared.at[pl.ds(PAD_ROWS, 16)], all_v)

    cols = [plsc.load_gather(all_v, [lane, jnp.full((16,), f, jnp.int32)])
            for f in range(NJ)]

    svec = jnp.zeros((16,), jnp.float32)
    for col in cols:
        svec = svec + col * col

    ibits = plsc.bitcast(svec, jnp.int32)
    g = plsc.bitcast(
        jnp.full((16,), 0x5F3759DF, jnp.int32)
        - lax.shift_right_logical(ibits, jnp.ones((16,), jnp.int32)),
        jnp.float32)
    half = 0.5 * svec
    for _ in range(3):
        g = g * (1.5 - half * g * g)
    t = jnp.maximum(svec * g, 1e-8)
    tinv = 1.0 / t

    neginf = jnp.float32(-jnp.inf)
    evec = jnp.zeros((16,), jnp.float32)
    for j in range(N):
        dj = _splat(cols[0], j) * cols[0]
        for col in cols[1:]:
            dj = dj + _splat(col, j) * col
        ej = jnp.max(jnp.where(lane < N, dj * tinv, neginf))
        evec = jnp.where(lane == j, lax.broadcast(ej, (16,)), evec)

    arow = jnp.zeros((16,), jnp.float32)
    for col in cols:
        arow = arow + _lane_bcast(col, jnp.full((16,), r, jnp.int32)) * col

    tr = _lane_bcast(t, jnp.full((16,), r, jnp.int32))
    anorm = jnp.where(lane < N, arow / (tr * evec), neginf)

    sk, sv = plsc.sort_key_val(anorm, lane, descending=True)
    vout_v[...] = sk
    iout_v[...] = sv
    aout_v[...] = anorm

    @pl.when((c == 0) & (s < N))
    def _():
        pltpu.sync_copy(vout_v, outv_hbm.at[s])
        pltpu.sync_copy(iout_v, outi_hbm.at[s])
        pltpu.sync_copy(aout_v, outa_hbm.at[s])


_sc_call = functools.partial(
    pl.kernel,
    out_type=(
        jax.ShapeDtypeStruct((N, 16), jnp.float32),
        jax.ShapeDtypeStruct((N, 16), jnp.int32),
        jax.ShapeDtypeStruct((N, 16), jnp.float32),
    ),
    mesh=plsc.VectorSubcoreMesh(core_axis_name="c", subcore_axis_name="s"),
    compiler_params=pltpu.CompilerParams(needs_layout_passes=False),
    scratch_types=[
        pltpu.VMEM((NFP,), jnp.float32),
        pltpu.VMEM((NJ, NFP), jnp.float32),
        pltpu.VMEM((16,), jnp.float32),
        pltpu.VMEM((16,), jnp.float32),
        pltpu.VMEM((16, 16), jnp.float32),
        pltpu.VMEM((16,), jnp.float32),
        pltpu.VMEM((16,), jnp.int32),
        pltpu.VMEM((16,), jnp.float32),
        pltpu.VMEM_SHARED((PAD_ROWS + 16, 16), jnp.float32),
    ],
)(_sc_body)


def kernel(x, W, b):
    x = x.reshape(x.shape[0], -1).astype(jnp.float32)
    xp = jnp.pad(x, ((0, 0), (0, NFP - NF)))
    wt = jnp.pad(W.astype(jnp.float32).T, ((0, 0), (0, NFP - NF)))
    bp = jnp.pad(b.astype(jnp.float32), (0, 16 - NJ))
    vals16, idx16, an16 = _sc_call(xp, wt, bp)
    vals = vals16[:, :N]
    idx = idx16[:, :N]
    an = an16[:, :N]
    row = jnp.repeat(jnp.arange(N, dtype=idx.dtype), N)
    edge_index = jnp.stack([row, idx.reshape(-1)])
    return (vals.reshape(-1), edge_index, an)


# speedup vs baseline: 1.1147x; 1.1147x over previous
import functools

import jax
import jax.numpy as jnp
from jax import lax
from jax.experimental import pallas as pl
from jax.experimental.pallas import tpu as pltpu
from jax.experimental.pallas import tpu_sc as plsc

N = 8
NF = 200
NFP = 208
NJ = 10
L = 16
NCHUNK = NFP // L
PAD_ROWS = 64

_GDN = lax.GatherDimensionNumbers(
    offset_dims=(), collapsed_slice_dims=(0,), start_index_map=(0,))


def _lane_bcast(vec, idx16):
    return lax.gather(vec, idx16[:, None], _GDN, (1,),
                      mode=lax.GatherScatterMode.PROMISE_IN_BOUNDS)


def _splat(vec, j):
    return _lane_bcast(vec, jnp.full((16,), j, jnp.int32))


def _sc_body(x_hbm, wt_hbm, b_hbm, outv_hbm, outi_hbm, outa_hbm,
             x_v, wt_v, b_v, stage_v, all_v, vout_v, iout_v, aout_v,
             sem1, sem2, sem3, ashared):
    c = lax.axis_index("c")
    s = lax.axis_index("s")
    lane = lax.iota(jnp.int32, 16)
    r = jnp.where(s < N, s, 0)
    active = (c == 0) & (s < N)

    @pl.when(active)
    def _():
        cp1 = pltpu.async_copy(wt_hbm, wt_v, sem1)
        cp2 = pltpu.async_copy(x_hbm.at[r], x_v, sem2)
        cp3 = pltpu.async_copy(b_hbm, b_v, sem3)
        cp1.wait()
        cp2.wait()
        cp3.wait()

        xc = [x_v[pl.ds(L * cc, L)] for cc in range(NCHUNK)]
        z = b_v[...]
        for j in range(NJ):
            acc = xc[0] * wt_v[j, pl.ds(0, L)]
            for cc in range(1, NCHUNK):
                acc = acc + xc[cc] * wt_v[j, pl.ds(L * cc, L)]
            z = jnp.where(lane == j, z + jnp.sum(acc), z)

        sig = 1.0 / (1.0 + jnp.exp(-z))
        atrr = jnp.where(lane < NJ, sig, 0.0)
        stage_v[...] = atrr
        pltpu.sync_copy(stage_v, ashared.at[PAD_ROWS + s])

    plsc.subcore_barrier()

    @pl.when(active)
    def _():
        pltpu.sync_copy(ashared.at[pl.ds(PAD_ROWS, 16)], all_v)

        cols = [plsc.load_gather(all_v, [lane, jnp.full((16,), f, jnp.int32)])
                for f in range(NJ)]

        svec = jnp.zeros((16,), jnp.float32)
        for col in cols:
            svec = svec + col * col

        ibits = plsc.bitcast(svec, jnp.int32)
        g = plsc.bitcast(
            jnp.full((16,), 0x5F3759DF, jnp.int32)
            - lax.shift_right_logical(ibits, jnp.ones((16,), jnp.int32)),
            jnp.float32)
        half = 0.5 * svec
        for _ in range(3):
            g = g * (1.5 - half * g * g)
        t = jnp.maximum(svec * g, 1e-8)
        tinv = 1.0 / t

        neginf = jnp.float32(-jnp.inf)
        evec = jnp.zeros((16,), jnp.float32)
        for j in range(N):
            dj = _splat(cols[0], j) * cols[0]
            for col in cols[1:]:
                dj = dj + _splat(col, j) * col
            ej = jnp.max(jnp.where(lane < N, dj * tinv, neginf))
            evec = jnp.where(lane == j, lax.broadcast(ej, (16,)), evec)

        arow = jnp.zeros((16,), jnp.float32)
        for col in cols:
            arow = arow + _lane_bcast(col, jnp.full((16,), r, jnp.int32)) * col

        tr = _lane_bcast(t, jnp.full((16,), r, jnp.int32))
        anorm = jnp.where(lane < N, arow / (tr * evec), neginf)

        sk, sv = plsc.sort_key_val(anorm, lane, descending=True)
        vout_v[...] = sk
        iout_v[...] = sv
        aout_v[...] = anorm

        pltpu.sync_copy(vout_v, outv_hbm.at[s])
        pltpu.sync_copy(iout_v, outi_hbm.at[s])
        pltpu.sync_copy(aout_v, outa_hbm.at[s])


_sc_call = functools.partial(
    pl.kernel,
    out_type=(
        jax.ShapeDtypeStruct((N, 16), jnp.float32),
        jax.ShapeDtypeStruct((N, 16), jnp.int32),
        jax.ShapeDtypeStruct((N, 16), jnp.float32),
    ),
    mesh=plsc.VectorSubcoreMesh(core_axis_name="c", subcore_axis_name="s",
                                num_cores=1),
    compiler_params=pltpu.CompilerParams(needs_layout_passes=False),
    scratch_types=[
        pltpu.VMEM((NFP,), jnp.float32),
        pltpu.VMEM((NJ, NFP), jnp.float32),
        pltpu.VMEM((16,), jnp.float32),
        pltpu.VMEM((16,), jnp.float32),
        pltpu.VMEM((16, 16), jnp.float32),
        pltpu.VMEM((16,), jnp.float32),
        pltpu.VMEM((16,), jnp.int32),
        pltpu.VMEM((16,), jnp.float32),
        pltpu.SemaphoreType.DMA,
        pltpu.SemaphoreType.DMA,
        pltpu.SemaphoreType.DMA,
        pltpu.VMEM_SHARED((PAD_ROWS + 16, 16), jnp.float32),
    ],
)(_sc_body)


def kernel(x, W, b):
    x = x.reshape(x.shape[0], -1).astype(jnp.float32)
    xp = jnp.pad(x, ((0, 0), (0, NFP - NF)))
    wt = jnp.pad(W.astype(jnp.float32).T, ((0, 0), (0, NFP - NF)))
    bp = jnp.pad(b.astype(jnp.float32), (0, 16 - NJ))
    vals16, idx16, an16 = _sc_call(xp, wt, bp)
    vals = vals16[:, :N]
    idx = idx16[:, :N]
    an = an16[:, :N]
    row = jnp.repeat(jnp.arange(N, dtype=idx.dtype), N)
    edge_index = jnp.stack([row, idx.reshape(-1)])
    return (vals.reshape(-1), edge_index, an)
